# Initial kernel scaffold; baseline (speedup 1.0000x reference)
import jax
import jax.numpy as jnp
from jax.experimental import pallas as pl


def kernel(atom, bond, edge_index, Wa1, ba1, ga, bta, Wa2, ba2, Wb1, bb1, gb, btb, Wb2, bb2):
    raise NotImplementedError("write your pallas kernel here")


# speedup vs baseline: 2.4923x; 2.4923x over previous
import functools

import jax
import jax.numpy as jnp
from jax import lax
from jax.experimental import pallas as pl
from jax.experimental.pallas import tpu as pltpu
from jax.experimental.pallas import tpu_sc as plsc

F32 = jnp.float32
EPS = 1e-5

NC = 2
NS = 16
NW = NC * NS
K = 80


def _zero_rows(ref, nrows, ncols):
    def body(r, _):
        for j in range(ncols // 16):
            ref[r, pl.ds(j * 16, 16)] = jnp.zeros((16,), F32)
        return 0
    lax.fori_loop(0, nrows, body, 0)


def _sc_gather_body(n_pad, d, ew, atom_hbm, src_hbm, dst_hbm,
                    sumh_hbm, part_hbm, sidx, didx, buf_a, buf_b, zbuf, acc, sem):
    c = lax.axis_index("c")
    s = lax.axis_index("s")
    wid = s * NC + c
    rows_per_tile = n_pad // NS
    zrows = zbuf.shape[0]

    _zero_rows(zbuf, zrows, d)
    for j in range(rows_per_tile // zrows):
        pltpu.sync_copy(zbuf, acc.at[pl.ds(s * rows_per_tile + j * zrows, zrows)])
    plsc.subcore_barrier()

    base = wid * ew

    def chunk(i, _):
        off = base + i * K
        pltpu.sync_copy(src_hbm.at[pl.ds(off, K)], sidx)
        pltpu.sync_copy(dst_hbm.at[pl.ds(off, K)], didx)
        ca = pltpu.async_copy(atom_hbm.at[sidx], buf_a, sem)
        cb = pltpu.async_copy(atom_hbm.at[didx], buf_b, sem)
        ca.wait()
        cb.wait()

        def addrow(r, _):
            for j in range(d // 16):
                sl = pl.ds(j * 16, 16)
                buf_b[r, sl] = buf_b[r, sl] + buf_a[r, sl]
            return 0
        lax.fori_loop(0, K, addrow, 0)

        pltpu.sync_copy(buf_b, sumh_hbm.at[pl.ds(off, K)])
        pltpu.sync_copy(buf_a, acc.at[didx], add=True)
        return 0

    lax.fori_loop(0, ew // K, chunk, 0)
    plsc.subcore_barrier()
    for j in range(rows_per_tile // zrows):
        r0 = s * rows_per_tile + j * zrows
        pltpu.sync_copy(acc.at[pl.ds(r0, zrows)], part_hbm.at[c, pl.ds(r0, zrows)])


def _sc_esum_body(n_pad, d, ew, e_hbm, dst_hbm, part_hbm,
                  didx, buf_a, zbuf, acc, sem):
    c = lax.axis_index("c")
    s = lax.axis_index("s")
    wid = s * NC + c
    rows_per_tile = n_pad // NS
    zrows = zbuf.shape[0]

    _zero_rows(zbuf, zrows, d)
    for j in range(rows_per_tile // zrows):
        pltpu.sync_copy(zbuf, acc.at[pl.ds(s * rows_per_tile + j * zrows, zrows)])
    plsc.subcore_barrier()

    base = wid * ew

    def chunk(i, _):
        off = base + i * K
        pltpu.sync_copy(dst_hbm.at[pl.ds(off, K)], didx)
        pltpu.sync_copy(e_hbm.at[pl.ds(off, K)], buf_a)
        pltpu.sync_copy(buf_a, acc.at[didx], add=True)
        return 0

    lax.fori_loop(0, ew // K, chunk, 0)
    plsc.subcore_barrier()
    for j in range(rows_per_tile // zrows):
        r0 = s * rows_per_tile + j * zrows
        pltpu.sync_copy(acc.at[pl.ds(r0, zrows)], part_hbm.at[c, pl.ds(r0, zrows)])


def _tc_edge1_body(sumh, bond, wt, wb, b1, y_out, st_out):
    y = jnp.dot(sumh[...], wt[...], preferred_element_type=F32)
    y = y + jnp.dot(bond[...], wb[...], preferred_element_type=F32)
    y = y + b1[...]
    y_out[...] = y
    s1 = jnp.sum(y, axis=0, keepdims=True)
    s2 = jnp.sum(y * y, axis=0, keepdims=True)
    st = jnp.concatenate([s1, s2], axis=0)

    @pl.when(pl.program_id(0) == 0)
    def _init():
        st_out[...] = jnp.zeros_like(st_out)

    st_out[...] += st


def _tc_edge2_body(n_edges, y_in, st, g, bt, w2, b2, e_out):
    inv_e = 1.0 / n_edges
    mu = st[0:1, :] * inv_e
    var = st[1:2, :] * inv_e - mu * mu
    a = g[...] * lax.rsqrt(var + EPS)
    cb = bt[...] - mu * a
    t = jnp.maximum(y_in[...] * a + cb, 0.0)
    e_out[...] = jnp.dot(t, w2[...], preferred_element_type=F32) + b2[...]


def _tc_node_body(n_nodes, ph, pe, wt, wb, b1, g, bt, w2, b2, h_out):
    hn = ph[0] + ph[1]
    en = pe[0] + pe[1]
    y = jnp.dot(hn, wt[...], preferred_element_type=F32)
    y = y + jnp.dot(en, wb[...], preferred_element_type=F32)
    y = y + b1[...]
    mu = jnp.mean(y, axis=0, keepdims=True)
    var = jnp.mean(y * y, axis=0, keepdims=True) - mu * mu
    t = jnp.maximum((y - mu) * lax.rsqrt(var + EPS) * g[...] + bt[...], 0.0)
    h_out[...] = jnp.dot(t, w2[...], preferred_element_type=F32) + b2[...]


def kernel(atom, bond, edge_index, Wa1, ba1, ga, bta, Wa2, ba2,
           Wb1, bb1, gb, btb, Wb2, bb2):
    n, d = atom.shape
    e_cnt, _ = bond.shape
    h_dim = Wb1.shape[1]
    ew = e_cnt // NW
    src = edge_index[0]
    dst = edge_index[1]

    mesh = plsc.VectorSubcoreMesh(core_axis_name="c", subcore_axis_name="s",
                                  num_cores=NC, num_subcores=NS)
    zrows = 128
    n_pad = NS * zrows * 5

    gather_call = pl.kernel(
        functools.partial(_sc_gather_body, n_pad, d, ew),
        out_type=(jax.ShapeDtypeStruct((e_cnt, d), F32),
                  jax.ShapeDtypeStruct((NC, n_pad, d), F32)),
        mesh=mesh,
        scratch_types=[
            pltpu.VMEM((K,), jnp.int32),
            pltpu.VMEM((K,), jnp.int32),
            pltpu.VMEM((K, d), F32),
            pltpu.VMEM((K, d), F32),
            pltpu.VMEM((zrows, d), F32),
            pltpu.VMEM_SHARED((n_pad, d), F32),
            pltpu.SemaphoreType.DMA,
        ],
    )
    sumh, part_h = gather_call(atom, src, dst)

    blk = 3200
    grid = e_cnt // blk
    y, stats = pl.pallas_call(
        _tc_edge1_body,
        grid=(grid,),
        in_specs=[
            pl.BlockSpec((blk, d), lambda i: (i, 0)),
            pl.BlockSpec((blk, d), lambda i: (i, 0)),
            pl.BlockSpec((d, h_dim), lambda i: (0, 0)),
            pl.BlockSpec((d, h_dim), lambda i: (0, 0)),
            pl.BlockSpec((1, h_dim), lambda i: (0, 0)),
        ],
        out_specs=[
            pl.BlockSpec((blk, h_dim), lambda i: (i, 0)),
            pl.BlockSpec((2, h_dim), lambda i: (0, 0)),
        ],
        out_shape=[
            jax.ShapeDtypeStruct((e_cnt, h_dim), F32),
            jax.ShapeDtypeStruct((2, h_dim), F32),
        ],
    )(sumh, bond, Wb1[:d], Wb1[d:], bb1.reshape(1, h_dim))

    e_out = pl.pallas_call(
        functools.partial(_tc_edge2_body, float(e_cnt)),
        grid=(grid,),
        in_specs=[
            pl.BlockSpec((blk, h_dim), lambda i: (i, 0)),
            pl.BlockSpec((2, h_dim), lambda i: (0, 0)),
            pl.BlockSpec((1, h_dim), lambda i: (0, 0)),
            pl.BlockSpec((1, h_dim), lambda i: (0, 0)),
            pl.BlockSpec((h_dim, d), lambda i: (0, 0)),
            pl.BlockSpec((1, d), lambda i: (0, 0)),
        ],
        out_specs=pl.BlockSpec((blk, d), lambda i: (i, 0)),
        out_shape=jax.ShapeDtypeStruct((e_cnt, d), F32),
    )(y, stats, gb.reshape(1, h_dim), btb.reshape(1, h_dim),
      Wb2, bb2.reshape(1, d))

    esum_call = pl.kernel(
        functools.partial(_sc_esum_body, n_pad, d, ew),
        out_type=jax.ShapeDtypeStruct((NC, n_pad, d), F32),
        mesh=mesh,
        scratch_types=[
            pltpu.VMEM((K,), jnp.int32),
            pltpu.VMEM((K, d), F32),
            pltpu.VMEM((zrows, d), F32),
            pltpu.VMEM_SHARED((n_pad, d), F32),
            pltpu.SemaphoreType.DMA,
        ],
    )
    part_e = esum_call(e_out, dst)

    h = pl.pallas_call(
        functools.partial(_tc_node_body, float(n)),
        grid=(1,),
        in_specs=[
            pl.BlockSpec((NC, n, d), lambda i: (0, 0, 0)),
            pl.BlockSpec((NC, n, d), lambda i: (0, 0, 0)),
            pl.BlockSpec((d, h_dim), lambda i: (0, 0)),
            pl.BlockSpec((d, h_dim), lambda i: (0, 0)),
            pl.BlockSpec((1, h_dim), lambda i: (0, 0)),
            pl.BlockSpec((1, h_dim), lambda i: (0, 0)),
            pl.BlockSpec((1, h_dim), lambda i: (0, 0)),
            pl.BlockSpec((h_dim, d), lambda i: (0, 0)),
            pl.BlockSpec((1, d), lambda i: (0, 0)),
        ],
        out_specs=pl.BlockSpec((n, d), lambda i: (0, 0)),
        out_shape=jax.ShapeDtypeStruct((n, d), F32),
    )(part_h, part_e, Wa1[:d], Wa1[d:], ba1.reshape(1, h_dim),
      ga.reshape(1, h_dim), bta.reshape(1, h_dim), Wa2, ba2.reshape(1, d))

    return h, e_out
